# Initial kernel scaffold; baseline (speedup 1.0000x reference)
import jax
import jax.numpy as jnp
from jax.experimental import pallas as pl


def kernel(x, edge_index, edge_weight, W1, b1, W2, b2, W_out, b_out):
    raise NotImplementedError("write your pallas kernel here")


# speedup vs baseline: 2.9522x; 2.9522x over previous
import functools

import jax
import jax.numpy as jnp
from jax import lax
from jax.experimental import pallas as pl
from jax.experimental.pallas import tpu as pltpu
from jax.experimental.pallas import tpu_sc as plsc

MIN_NORM = 1e-15
EPS = 4e-3
MAXNORM = 1.0 - EPS

_NC = 2
_NS = 16
_CH = 128


def _artanh(x):
    x = jnp.clip(x, -1.0 + 1e-7, 1.0 - 1e-7)
    return 0.5 * jnp.log((1.0 + x) / (1.0 - x))


def _norm(x):
    return jnp.maximum(
        jnp.sqrt(jnp.sum(x * x, axis=-1, keepdims=True)), MIN_NORM)


def _proj(x):
    n = _norm(x)
    return jnp.where(n > MAXNORM, x / n * MAXNORM, x)


def _expmap0(u):
    n = _norm(u)
    return jnp.tanh(n) * u / n


def _logmap0(p):
    n = _norm(p)
    return _artanh(n) * p / n


def _mobius_add(x, y):
    x2 = jnp.sum(x * x, axis=-1, keepdims=True)
    y2 = jnp.sum(y * y, axis=-1, keepdims=True)
    xy = jnp.sum(x * y, axis=-1, keepdims=True)
    num = (1.0 + 2.0 * xy + y2) * x + (1.0 - x2) * y
    denom = 1.0 + 2.0 * xy + x2 * y2
    return num / jnp.maximum(denom, MIN_NORM)


def _mobius_matvec(h, W):
    mx = lax.dot_general(h, W, dimension_numbers=(((1,), (1,)), ((), ())),
                         preferred_element_type=jnp.float32)
    xn = _norm(h)
    mxn = _norm(mx)
    res = jnp.tanh(mxn / xn * _artanh(xn)) * mx / mxn
    cond = jnp.all(mx == 0.0, axis=-1, keepdims=True)
    return jnp.where(cond, jnp.zeros_like(res), res)


def _hyp_linear(h, W, b):
    mv = _proj(_mobius_matvec(h, W))
    hyp_bias = _proj(_expmap0(b))
    return _proj(_mobius_add(mv, hyp_bias))


def _dense1_body(x_ref, W_ref, b_ref, out_ref):
    res = _hyp_linear(x_ref[...], W_ref[...], b_ref[...])
    out_ref[...] = _logmap0(res)


def _mid_body(p_ref, W_ref, b_ref, out_ref):
    agg = p_ref[0] + p_ref[1]
    h2 = _proj(_expmap0(agg))
    xt2 = jnp.maximum(_logmap0(h2), 0.0)
    h = _proj(_expmap0(xt2))
    res = _hyp_linear(h, W_ref[...], b_ref[...])
    xt = _logmap0(res)
    d = xt.shape[-1] // 2
    out_ref[0] = xt[:, :d]
    out_ref[1] = xt[:, d:]


def _final_body(p_ref, W_ref, b_ref, out_ref):
    agg = jnp.concatenate([p_ref[0], p_ref[1]], axis=-1)
    h2 = _proj(_expmap0(agg))
    xt2 = _logmap0(h2)
    h = _proj(_expmap0(xt2))
    ht = _logmap0(h)
    out = lax.dot_general(ht, W_ref[...],
                          dimension_numbers=(((1,), (1,)), ((), ())),
                          preferred_element_type=jnp.float32)
    out_ref[...] = out + b_ref[...]


def _row_blocked(body, n, br, in_specs, out_shape, out_spec):
    grid = (n // br,)
    return pl.pallas_call(
        body,
        grid=grid,
        in_specs=in_specs,
        out_specs=out_spec,
        out_shape=out_shape,
    )


def _make_agg(n, npad, d, nchunks_tile, per_core_table):
    zrows = npad // _NS
    assert zrows % _CH == 0
    nfull = zrows // _CH
    mesh = plsc.VectorSubcoreMesh(core_axis_name="c", subcore_axis_name="s",
                                  num_cores=_NC, num_subcores=_NS)

    def body(table, src2, dst2, w2, out, acc, sidx, didx, wv, rows, sem):
        c = lax.axis_index("c")
        s = lax.axis_index("s")

        def zero_row(e, carry):
            for k in range(d // 16):
                rows[e, pl.ds(k * 16, 16)] = jnp.zeros((16,), jnp.float32)
            return carry

        lax.fori_loop(0, _CH, zero_row, 0)
        r0 = s * zrows
        for q in range(nfull):
            pltpu.sync_copy(rows, acc.at[pl.ds(r0 + q * _CH, _CH)])
        plsc.subcore_barrier()

        if per_core_table:
            cb = s * nchunks_tile
        else:
            cb = (c * _NS + s) * nchunks_tile
        pltpu.sync_copy(src2.at[pl.ds(cb, nchunks_tile)], sidx)
        pltpu.sync_copy(dst2.at[pl.ds(cb, nchunks_tile)], didx)
        pltpu.sync_copy(w2.at[pl.ds(cb, nchunks_tile)], wv)

        tix = c if per_core_table else 0

        def chunk(j, carry):
            pltpu.async_copy(table.at[tix].at[sidx.at[j]], rows, sem).wait()

            def scale(g, carry2):
                wvec = wv[j, pl.ds(g * 16, 16)]
                for el in range(16):
                    e = g * 16 + el
                    wsc = wvec[el]
                    for k in range(d // 16):
                        sl = pl.ds(k * 16, 16)
                        rows[e, sl] = rows[e, sl] * wsc
                return carry2

            lax.fori_loop(0, _CH // 16, scale, 0)

            pltpu.sync_copy(rows, acc.at[didx.at[j]], add=True)
            return carry

        lax.fori_loop(0, nchunks_tile, chunk, 0)
        plsc.subcore_barrier()

        pltpu.sync_copy(acc.at[pl.ds(r0, zrows)],
                        out.at[c].at[pl.ds(r0, zrows)])

    return pl.kernel(
        body,
        out_type=jax.ShapeDtypeStruct((_NC, npad, d), jnp.float32),
        mesh=mesh,
        scratch_types=[
            pltpu.VMEM_SHARED((npad, d), jnp.float32),
            pltpu.VMEM((nchunks_tile, _CH), jnp.int32),
            pltpu.VMEM((nchunks_tile, _CH), jnp.int32),
            pltpu.VMEM((nchunks_tile, _CH), jnp.float32),
            pltpu.VMEM((_CH, d), jnp.float32),
            pltpu.SemaphoreType.DMA,
        ],
    )


def kernel(x, edge_index, edge_weight, W1, b1, W2, b2, W_out, b_out):
    n, d_lat = x.shape
    e = edge_weight.shape[0]
    d_hid = W1.shape[0]
    d_feat = W2.shape[0]

    align = _NC * _NS * _CH
    epad = ((e + align - 1) // align) * align
    pad = epad - e
    src = edge_index[0].astype(jnp.int32)
    dst = edge_index[1].astype(jnp.int32)
    w = edge_weight.astype(jnp.float32)
    if pad:
        zi = jnp.zeros((pad,), jnp.int32)
        src = jnp.concatenate([src, zi])
        dst = jnp.concatenate([dst, zi])
        w = jnp.concatenate([w, jnp.zeros((pad,), jnp.float32)])
    src2 = src.reshape(epad // _CH, _CH)
    dst2 = dst.reshape(epad // _CH, _CH)
    w2 = w.reshape(epad // _CH, _CH)

    br = 2000
    b1r = b1.reshape(1, d_hid)
    b2r = b2.reshape(1, d_feat)
    b_outr = b_out.reshape(1, -1)

    full = lambda shape: pl.BlockSpec(shape, lambda i: tuple(0 for _ in shape))

    xt1 = _row_blocked(
        _dense1_body, n, br,
        [pl.BlockSpec((br, d_lat), lambda i: (i, 0)),
         full((d_hid, d_lat)), full((1, d_hid))],
        jax.ShapeDtypeStruct((n, d_hid), jnp.float32),
        pl.BlockSpec((br, d_hid), lambda i: (i, 0)),
    )(x, W1, b1r)

    ralign = _NS * _CH
    npad = ((n + ralign - 1) // ralign) * ralign
    agg1 = _make_agg(n, npad, d_hid, epad // _CH // (_NC * _NS), False)(
        xt1.reshape(1, n, d_hid), src2, dst2, w2)

    xt2h = _row_blocked(
        _mid_body, n, br,
        [pl.BlockSpec((2, br, d_hid), lambda i: (0, i, 0)),
         full((d_feat, d_hid)), full((1, d_feat))],
        jax.ShapeDtypeStruct((2, n, d_feat // 2), jnp.float32),
        pl.BlockSpec((2, br, d_feat // 2), lambda i: (0, i, 0)),
    )(agg1, W2, b2r)

    agg2 = _make_agg(n, npad, d_feat // 2, epad // _CH // _NS, True)(
        xt2h, src2, dst2, w2)

    n_out = W_out.shape[0]
    out = _row_blocked(
        _final_body, n, br,
        [pl.BlockSpec((2, br, d_feat // 2), lambda i: (0, i, 0)),
         full((n_out, d_feat)), full((1, n_out))],
        jax.ShapeDtypeStruct((n, n_out), jnp.float32),
        pl.BlockSpec((br, n_out), lambda i: (i, 0)),
    )(agg2, W_out, b_outr)

    return out


# speedup vs baseline: 3.2845x; 1.1126x over previous
import functools

import jax
import jax.numpy as jnp
from jax import lax
from jax.experimental import pallas as pl
from jax.experimental.pallas import tpu as pltpu
from jax.experimental.pallas import tpu_sc as plsc

MIN_NORM = 1e-15
EPS = 4e-3
MAXNORM = 1.0 - EPS

_NC = 2
_NS = 16
_CH = 128
_G = 10


def _artanh(x):
    x = jnp.clip(x, -1.0 + 1e-7, 1.0 - 1e-7)
    return 0.5 * jnp.log((1.0 + x) / (1.0 - x))


def _norm(x):
    return jnp.maximum(
        jnp.sqrt(jnp.sum(x * x, axis=-1, keepdims=True)), MIN_NORM)


def _proj(x):
    n = _norm(x)
    return jnp.where(n > MAXNORM, x / n * MAXNORM, x)


def _expmap0(u):
    n = _norm(u)
    return jnp.tanh(n) * u / n


def _logmap0(p):
    n = _norm(p)
    return _artanh(n) * p / n


def _mobius_add(x, y):
    x2 = jnp.sum(x * x, axis=-1, keepdims=True)
    y2 = jnp.sum(y * y, axis=-1, keepdims=True)
    xy = jnp.sum(x * y, axis=-1, keepdims=True)
    num = (1.0 + 2.0 * xy + y2) * x + (1.0 - x2) * y
    denom = 1.0 + 2.0 * xy + x2 * y2
    return num / jnp.maximum(denom, MIN_NORM)


def _mobius_matvec(h, W):
    mx = lax.dot_general(h, W, dimension_numbers=(((1,), (1,)), ((), ())),
                         preferred_element_type=jnp.float32)
    xn = _norm(h)
    mxn = _norm(mx)
    res = jnp.tanh(mxn / xn * _artanh(xn)) * mx / mxn
    cond = jnp.all(mx == 0.0, axis=-1, keepdims=True)
    return jnp.where(cond, jnp.zeros_like(res), res)


def _hyp_linear(h, W, b):
    mv = _proj(_mobius_matvec(h, W))
    hyp_bias = _proj(_expmap0(b))
    return _proj(_mobius_add(mv, hyp_bias))


def _dense1_body(x_ref, W_ref, b_ref, out_ref):
    res = _hyp_linear(x_ref[...], W_ref[...], b_ref[...])
    out_ref[...] = _logmap0(res)


def _mid_body(p_ref, W_ref, b_ref, out_ref):
    agg = p_ref[0] + p_ref[1]
    h2 = _proj(_expmap0(agg))
    u = jnp.maximum(_logmap0(h2), 0.0)
    mu = lax.dot_general(u, W_ref[...],
                         dimension_numbers=(((1,), (1,)), ((), ())),
                         preferred_element_type=jnp.float32)
    mv = _proj(_expmap0(mu))
    hyp_bias = _proj(_expmap0(b_ref[...]))
    res = _proj(_mobius_add(mv, hyp_bias))
    xt = _logmap0(res)
    d = xt.shape[-1] // 2
    out_ref[0] = xt[:, :d]
    out_ref[1] = xt[:, d:]


def _final_body(p_ref, W_ref, b_ref, out_ref):
    agg = jnp.concatenate([p_ref[0], p_ref[1]], axis=-1)
    ht = _logmap0(_proj(_expmap0(agg)))
    out = lax.dot_general(ht, W_ref[...],
                          dimension_numbers=(((1,), (1,)), ((), ())),
                          preferred_element_type=jnp.float32)
    out_ref[...] = out + b_ref[...]


def _row_blocked(body, n, br, in_specs, out_shape, out_spec):
    grid = (n // br,)
    return pl.pallas_call(
        body,
        grid=grid,
        in_specs=in_specs,
        out_specs=out_spec,
        out_shape=out_shape,
    )


def _make_agg(n, npad, d, nchunks_tile, per_core_table):
    zrows = npad // _NS
    assert zrows % _CH == 0
    nfull = zrows // _CH
    mesh = plsc.VectorSubcoreMesh(core_axis_name="c", subcore_axis_name="s",
                                  num_cores=_NC, num_subcores=_NS)

    assert nchunks_tile % _G == 0
    ngroups = nchunks_tile // _G
    assert ngroups % 2 == 0

    def body(table, src3, dst3, w3, out, acc,
             sb0, sb1, db0, db1, wb0, wb1, rows0, rows1, *sems):
        c = lax.axis_index("c")
        s = lax.axis_index("s")
        rowbufs = (rows0, rows1)
        sbufs = (sb0, sb1)
        dbufs = (db0, db1)
        wbufs = (wb0, wb1)
        gsems = sems[0:2]
        ssems = sems[2:4]
        isems = sems[4:6]

        def zero_row(e, carry):
            for k in range(d // 16):
                rows0[e, pl.ds(k * 16, 16)] = jnp.zeros((16,), jnp.float32)
            return carry

        lax.fori_loop(0, _CH, zero_row, 0)
        r0 = s * zrows
        for q in range(nfull):
            pltpu.sync_copy(rows0, acc.at[pl.ds(r0 + q * _CH, _CH)])
        plsc.subcore_barrier()

        if per_core_table:
            g0 = s * ngroups
        else:
            g0 = (c * _NS + s) * ngroups
        tix = c if per_core_table else 0

        def stage_issue(gg, p):
            pltpu.async_copy(src3.at[g0 + gg], sbufs[p], isems[p])
            pltpu.async_copy(dst3.at[g0 + gg], dbufs[p], isems[p])
            pltpu.async_copy(w3.at[g0 + gg], wbufs[p], isems[p])

        def stage_wait(p):
            pltpu.make_async_copy(src3.at[g0], sbufs[p], isems[p]).wait()
            pltpu.make_async_copy(dst3.at[g0], dbufs[p], isems[p]).wait()
            pltpu.make_async_copy(w3.at[g0], wbufs[p], isems[p]).wait()

        def g_issue(p, b, rb):
            pltpu.async_copy(table.at[tix].at[sbufs[p].at[b]], rowbufs[rb],
                             gsems[rb])

        def g_wait(rb):
            pltpu.make_async_copy(table.at[tix].at[sb0.at[0]], rowbufs[rb],
                                  gsems[rb]).wait()

        def s_issue(p, b, rb):
            pltpu.async_copy(rowbufs[rb], acc.at[dbufs[p].at[b]], ssems[rb],
                             add=True)

        def s_wait(rb):
            pltpu.make_async_copy(rowbufs[rb], acc.at[db0.at[0]],
                                  ssems[rb]).wait()

        def scale_buf(p, b, buf):
            def scale(g, carry2):
                wvec = wbufs[p][b, pl.ds(g * 16, 16)]
                for el in range(16):
                    e = g * 16 + el
                    wsc = wvec[el]
                    for k in range(d // 16):
                        sl = pl.ds(k * 16, 16)
                        buf[e, sl] = buf[e, sl] * wsc
                return carry2

            lax.fori_loop(0, _CH // 16, scale, 0)

        pltpu.sync_copy(src3.at[g0], sb0)
        pltpu.sync_copy(dst3.at[g0], db0)
        pltpu.sync_copy(w3.at[g0], wb0)
        g_issue(0, 0, 0)

        def grouppair(jj2, carry):
            for p in range(2):
                g = jj2 * 2 + p

                @pl.when(g + 1 < ngroups)
                def _stage():
                    stage_issue(g + 1, 1 - p)
                for b in range(_G):
                    j = g * _G + b
                    rb = b & 1
                    g_wait(rb)

                    @pl.when(j >= 1)
                    def _drain():
                        s_wait(1 - rb)
                    if b + 1 < _G:
                        g_issue(p, b + 1, 1 - rb)
                    else:
                        @pl.when(g + 1 < ngroups)
                        def _next():
                            stage_wait(1 - p)
                            g_issue(1 - p, 0, 1 - rb)
                    scale_buf(p, b, rowbufs[rb])
                    s_issue(p, b, rb)
            return carry

        lax.fori_loop(0, ngroups // 2, grouppair, 0)
        s_wait((nchunks_tile - 1) & 1)
        plsc.subcore_barrier()

        pltpu.sync_copy(acc.at[pl.ds(r0, zrows)],
                        out.at[c].at[pl.ds(r0, zrows)])

    return pl.kernel(
        body,
        out_type=jax.ShapeDtypeStruct((_NC, npad, d), jnp.float32),
        mesh=mesh,
        scratch_types=[
            pltpu.VMEM_SHARED((npad, d), jnp.float32),
            pltpu.VMEM((_G, _CH), jnp.int32),
            pltpu.VMEM((_G, _CH), jnp.int32),
            pltpu.VMEM((_G, _CH), jnp.int32),
            pltpu.VMEM((_G, _CH), jnp.int32),
            pltpu.VMEM((_G, _CH), jnp.float32),
            pltpu.VMEM((_G, _CH), jnp.float32),
            pltpu.VMEM((_CH, d), jnp.float32),
            pltpu.VMEM((_CH, d), jnp.float32),
        ] + [pltpu.SemaphoreType.DMA] * 6,
    )


def kernel(x, edge_index, edge_weight, W1, b1, W2, b2, W_out, b_out):
    n, d_lat = x.shape
    e = edge_weight.shape[0]
    d_hid = W1.shape[0]
    d_feat = W2.shape[0]

    align = _NC * _NS * _CH * 2 * _G
    epad = ((e + align - 1) // align) * align
    pad = epad - e
    src = edge_index[0].astype(jnp.int32)
    dst = edge_index[1].astype(jnp.int32)
    w = edge_weight.astype(jnp.float32)
    if pad:
        zi = jnp.zeros((pad,), jnp.int32)
        src = jnp.concatenate([src, zi])
        dst = jnp.concatenate([dst, zi])
        w = jnp.concatenate([w, jnp.zeros((pad,), jnp.float32)])
    src2 = src.reshape(-1, _G, _CH)
    dst2 = dst.reshape(-1, _G, _CH)
    w2 = w.reshape(-1, _G, _CH)

    br = 2000
    b1r = b1.reshape(1, d_hid)
    b2r = b2.reshape(1, d_feat)
    b_outr = b_out.reshape(1, -1)

    full = lambda shape: pl.BlockSpec(shape, lambda i: tuple(0 for _ in shape))

    xt1 = _row_blocked(
        _dense1_body, n, br,
        [pl.BlockSpec((br, d_lat), lambda i: (i, 0)),
         full((d_hid, d_lat)), full((1, d_hid))],
        jax.ShapeDtypeStruct((n, d_hid), jnp.float32),
        pl.BlockSpec((br, d_hid), lambda i: (i, 0)),
    )(x, W1, b1r)

    ralign = _NS * _CH
    npad = ((n + ralign - 1) // ralign) * ralign
    agg1 = _make_agg(n, npad, d_hid, epad // _CH // (_NC * _NS), False)(
        xt1.reshape(1, n, d_hid), src2, dst2, w2)

    xt2h = _row_blocked(
        _mid_body, n, br,
        [pl.BlockSpec((2, br, d_hid), lambda i: (0, i, 0)),
         full((d_feat, d_hid)), full((1, d_feat))],
        jax.ShapeDtypeStruct((2, n, d_feat // 2), jnp.float32),
        pl.BlockSpec((2, br, d_feat // 2), lambda i: (0, i, 0)),
    )(agg1, W2, b2r)

    agg2 = _make_agg(n, npad, d_feat // 2, epad // _CH // _NS, True)(
        xt2h, src2, dst2, w2)

    n_out = W_out.shape[0]
    out = _row_blocked(
        _final_body, n, br,
        [pl.BlockSpec((2, br, d_feat // 2), lambda i: (0, i, 0)),
         full((n_out, d_feat)), full((1, n_out))],
        jax.ShapeDtypeStruct((n, n_out), jnp.float32),
        pl.BlockSpec((br, n_out), lambda i: (i, 0)),
    )(agg2, W_out, b_outr)

    return out


# speedup vs baseline: 3.6855x; 1.1221x over previous
import functools

import jax
import jax.numpy as jnp
from jax import lax
from jax.experimental import pallas as pl
from jax.experimental.pallas import tpu as pltpu
from jax.experimental.pallas import tpu_sc as plsc

MIN_NORM = 1e-15
EPS = 4e-3
MAXNORM = 1.0 - EPS

_NC = 2
_NS = 16
_CH = 64
_G = 10


def _artanh(x):
    x = jnp.clip(x, -1.0 + 1e-7, 1.0 - 1e-7)
    return 0.5 * jnp.log((1.0 + x) / (1.0 - x))


def _norm(x):
    return jnp.maximum(
        jnp.sqrt(jnp.sum(x * x, axis=-1, keepdims=True)), MIN_NORM)


def _proj(x):
    n = _norm(x)
    return jnp.where(n > MAXNORM, x / n * MAXNORM, x)


def _expmap0(u):
    n = _norm(u)
    return jnp.tanh(n) * u / n


def _logmap0(p):
    n = _norm(p)
    return _artanh(n) * p / n


def _mobius_add(x, y):
    x2 = jnp.sum(x * x, axis=-1, keepdims=True)
    y2 = jnp.sum(y * y, axis=-1, keepdims=True)
    xy = jnp.sum(x * y, axis=-1, keepdims=True)
    num = (1.0 + 2.0 * xy + y2) * x + (1.0 - x2) * y
    denom = 1.0 + 2.0 * xy + x2 * y2
    return num / jnp.maximum(denom, MIN_NORM)


def _mobius_matvec(h, W):
    mx = lax.dot_general(h, W, dimension_numbers=(((1,), (1,)), ((), ())),
                         preferred_element_type=jnp.float32)
    xn = _norm(h)
    mxn = _norm(mx)
    res = jnp.tanh(mxn / xn * _artanh(xn)) * mx / mxn
    cond = jnp.all(mx == 0.0, axis=-1, keepdims=True)
    return jnp.where(cond, jnp.zeros_like(res), res)


def _hyp_linear(h, W, b):
    mv = _proj(_mobius_matvec(h, W))
    hyp_bias = _proj(_expmap0(b))
    return _proj(_mobius_add(mv, hyp_bias))


def _dense1_body(x_ref, W_ref, b_ref, out_ref):
    res = _hyp_linear(x_ref[...], W_ref[...], b_ref[...])
    out_ref[...] = _logmap0(res)


def _mid_body(p_ref, W_ref, b_ref, out_ref):
    agg = p_ref[0] + p_ref[1]
    h2 = _proj(_expmap0(agg))
    u = jnp.maximum(_logmap0(h2), 0.0)
    mu = lax.dot_general(u, W_ref[...],
                         dimension_numbers=(((1,), (1,)), ((), ())),
                         preferred_element_type=jnp.float32)
    mv = _proj(_expmap0(mu))
    hyp_bias = _proj(_expmap0(b_ref[...]))
    res = _proj(_mobius_add(mv, hyp_bias))
    xt = _logmap0(res)
    d = xt.shape[-1] // 2
    out_ref[0] = xt[:, :d]
    out_ref[1] = xt[:, d:]


def _final_body(p_ref, W_ref, b_ref, out_ref):
    agg = jnp.concatenate([p_ref[0], p_ref[1]], axis=-1)
    ht = _logmap0(_proj(_expmap0(agg)))
    out = lax.dot_general(ht, W_ref[...],
                          dimension_numbers=(((1,), (1,)), ((), ())),
                          preferred_element_type=jnp.float32)
    out_ref[...] = out + b_ref[...]


def _row_blocked(body, n, br, in_specs, out_shape, out_spec):
    grid = (n // br,)
    return pl.pallas_call(
        body,
        grid=grid,
        in_specs=in_specs,
        out_specs=out_spec,
        out_shape=out_shape,
    )


def _make_agg(n, npad, d, nchunks_tile, per_core_table):
    zrows = npad // _NS
    assert zrows % _CH == 0
    nfull = zrows // _CH
    mesh = plsc.VectorSubcoreMesh(core_axis_name="c", subcore_axis_name="s",
                                  num_cores=_NC, num_subcores=_NS)

    assert nchunks_tile % _G == 0
    ngroups = nchunks_tile // _G
    assert ngroups % 2 == 0

    def body(table, src3, dst3, w3, out, acc,
             sb0, sb1, db0, db1, wb0, wb1,
             rows0, rows1, rows2, rows3, *sems):
        c = lax.axis_index("c")
        s = lax.axis_index("s")
        rowbufs = (rows0, rows1, rows2, rows3)
        sbufs = (sb0, sb1)
        dbufs = (db0, db1)
        wbufs = (wb0, wb1)
        gsems = sems[0:4]
        ssems = sems[4:8]
        isems = sems[8:10]

        def zero_row(e, carry):
            for k in range(d // 16):
                rows0[e, pl.ds(k * 16, 16)] = jnp.zeros((16,), jnp.float32)
            return carry

        lax.fori_loop(0, _CH, zero_row, 0)
        r0 = s * zrows
        for q in range(nfull):
            pltpu.sync_copy(rows0, acc.at[pl.ds(r0 + q * _CH, _CH)])
        plsc.subcore_barrier()

        if per_core_table:
            g0 = s * ngroups
        else:
            g0 = (c * _NS + s) * ngroups
        tix = c if per_core_table else 0

        def stage_issue(gg, p):
            pltpu.async_copy(src3.at[g0 + gg], sbufs[p], isems[p])
            pltpu.async_copy(dst3.at[g0 + gg], dbufs[p], isems[p])
            pltpu.async_copy(w3.at[g0 + gg], wbufs[p], isems[p])

        def stage_wait(p):
            pltpu.make_async_copy(src3.at[g0], sbufs[p], isems[p]).wait()
            pltpu.make_async_copy(dst3.at[g0], dbufs[p], isems[p]).wait()
            pltpu.make_async_copy(w3.at[g0], wbufs[p], isems[p]).wait()

        def g_issue(p, b, rb):
            pltpu.async_copy(table.at[tix].at[sbufs[p].at[b]], rowbufs[rb],
                             gsems[rb])

        def g_wait(rb):
            pltpu.make_async_copy(table.at[tix].at[sb0.at[0]], rowbufs[rb],
                                  gsems[rb]).wait()

        def s_issue(p, b, rb):
            pltpu.async_copy(rowbufs[rb], acc.at[dbufs[p].at[b]], ssems[rb],
                             add=True)

        def s_wait(rb):
            pltpu.make_async_copy(rowbufs[rb], acc.at[db0.at[0]],
                                  ssems[rb]).wait()

        def scale_buf(p, b, buf):
            def scale(g, carry2):
                wvec = wbufs[p][b, pl.ds(g * 16, 16)]
                for el in range(16):
                    e = g * 16 + el
                    wsc = wvec[el]
                    for k in range(d // 16):
                        sl = pl.ds(k * 16, 16)
                        buf[e, sl] = buf[e, sl] * wsc
                return carry2

            lax.fori_loop(0, _CH // 16, scale, 0)

        pltpu.sync_copy(src3.at[g0], sb0)
        pltpu.sync_copy(dst3.at[g0], db0)
        pltpu.sync_copy(w3.at[g0], wb0)
        g_issue(0, 0, 0)
        g_issue(0, 1, 1)

        def grouppair(jj2, carry):
            for p in range(2):
                g = jj2 * 2 + p

                @pl.when(g + 1 < ngroups)
                def _stage():
                    stage_issue(g + 1, 1 - p)
                for b in range(_G):
                    j = g * _G + b
                    rb = (2 * p + b) % 4
                    g_wait(rb)
                    scale_buf(p, b, rowbufs[rb])
                    s_issue(p, b, rb)
                    bn = (rb + 2) % 4
                    if b == _G - 2:
                        @pl.when(g + 1 < ngroups)
                        def _nextstage():
                            stage_wait(1 - p)

                    @pl.when(j + 2 < nchunks_tile)
                    def _prefetch():
                        @pl.when(j + 2 >= 4)
                        def _drain():
                            s_wait(bn)
                        if b + 2 < _G:
                            g_issue(p, b + 2, bn)
                        else:
                            g_issue(1 - p, b + 2 - _G, bn)
            return carry

        lax.fori_loop(0, ngroups // 2, grouppair, 0)
        for b in range(4):
            s_wait(b)
        plsc.subcore_barrier()

        pltpu.sync_copy(acc.at[pl.ds(r0, zrows)],
                        out.at[c].at[pl.ds(r0, zrows)])

    return pl.kernel(
        body,
        out_type=jax.ShapeDtypeStruct((_NC, npad, d), jnp.float32),
        mesh=mesh,
        scratch_types=[
            pltpu.VMEM_SHARED((npad, d), jnp.float32),
            pltpu.VMEM((_G, _CH), jnp.int32),
            pltpu.VMEM((_G, _CH), jnp.int32),
            pltpu.VMEM((_G, _CH), jnp.int32),
            pltpu.VMEM((_G, _CH), jnp.int32),
            pltpu.VMEM((_G, _CH), jnp.float32),
            pltpu.VMEM((_G, _CH), jnp.float32),
            pltpu.VMEM((_CH, d), jnp.float32),
            pltpu.VMEM((_CH, d), jnp.float32),
            pltpu.VMEM((_CH, d), jnp.float32),
            pltpu.VMEM((_CH, d), jnp.float32),
        ] + [pltpu.SemaphoreType.DMA] * 10,
    )


def kernel(x, edge_index, edge_weight, W1, b1, W2, b2, W_out, b_out):
    n, d_lat = x.shape
    e = edge_weight.shape[0]
    d_hid = W1.shape[0]
    d_feat = W2.shape[0]

    align = _NC * _NS * _CH * 2 * _G
    epad = ((e + align - 1) // align) * align
    pad = epad - e
    src = edge_index[0].astype(jnp.int32)
    dst = edge_index[1].astype(jnp.int32)
    w = edge_weight.astype(jnp.float32)
    if pad:
        zi = jnp.zeros((pad,), jnp.int32)
        src = jnp.concatenate([src, zi])
        dst = jnp.concatenate([dst, zi])
        w = jnp.concatenate([w, jnp.zeros((pad,), jnp.float32)])
    src2 = src.reshape(-1, _G, _CH)
    dst2 = dst.reshape(-1, _G, _CH)
    w2 = w.reshape(-1, _G, _CH)

    br = 2000
    b1r = b1.reshape(1, d_hid)
    b2r = b2.reshape(1, d_feat)
    b_outr = b_out.reshape(1, -1)

    full = lambda shape: pl.BlockSpec(shape, lambda i: tuple(0 for _ in shape))

    xt1 = _row_blocked(
        _dense1_body, n, br,
        [pl.BlockSpec((br, d_lat), lambda i: (i, 0)),
         full((d_hid, d_lat)), full((1, d_hid))],
        jax.ShapeDtypeStruct((n, d_hid), jnp.float32),
        pl.BlockSpec((br, d_hid), lambda i: (i, 0)),
    )(x, W1, b1r)

    ralign = _NS * _CH
    npad = ((n + ralign - 1) // ralign) * ralign
    agg1 = _make_agg(n, npad, d_hid, epad // _CH // (_NC * _NS), False)(
        xt1.reshape(1, n, d_hid), src2, dst2, w2)

    xt2h = _row_blocked(
        _mid_body, n, br,
        [pl.BlockSpec((2, br, d_hid), lambda i: (0, i, 0)),
         full((d_feat, d_hid)), full((1, d_feat))],
        jax.ShapeDtypeStruct((2, n, d_feat // 2), jnp.float32),
        pl.BlockSpec((2, br, d_feat // 2), lambda i: (0, i, 0)),
    )(agg1, W2, b2r)

    agg2 = _make_agg(n, npad, d_feat // 2, epad // _CH // _NS, True)(
        xt2h, src2, dst2, w2)

    n_out = W_out.shape[0]
    out = _row_blocked(
        _final_body, n, br,
        [pl.BlockSpec((2, br, d_feat // 2), lambda i: (0, i, 0)),
         full((n_out, d_feat)), full((1, n_out))],
        jax.ShapeDtypeStruct((n, n_out), jnp.float32),
        pl.BlockSpec((br, n_out), lambda i: (i, 0)),
    )(agg2, W_out, b_outr)

    return out
